# baseline (speedup 1.0000x reference)
import functools

import jax
import jax.numpy as jnp
from jax import lax
from jax.experimental import pallas as pl
from jax.experimental.pallas import tpu as pltpu
from jax.experimental.pallas import tpu_sc as plsc

N = 4096
M = 4096
NNZ = 163840

NC = 2
NS = 16
L = 16
CHUNK = NNZ // NS


def _sc_body(pd, pr, pc, ad, ar, ac, x, y, sv, w,
             opx, opux, oatuy, oaux,
             dat, ia, ib, srcA, srcB, t1, t2, acc0, acc1, sem):
    c = lax.axis_index("c")
    s = lax.axis_index("s")
    off = s * CHUNK

    HALF = CHUNK // 2

    @plsc.parallel_loop(0, N, L, unroll=4)
    def _z(i):
        z = jnp.zeros((L,), jnp.float32)
        acc0[pl.ds(i, L)] = z
        acc1[pl.ds(i, L)] = z

    @pl.when(c == 0)
    def _p_side():
        h0 = pl.ds(0, HALF)
        h1 = pl.ds(HALF, HALF)
        first = [
            pltpu.async_copy(x, srcA, sem),
            pltpu.async_copy(w.at[pl.ds(0, N)], srcB, sem),
            pltpu.async_copy(pd.at[pl.ds(off, HALF)], dat.at[h0], sem),
            pltpu.async_copy(pr.at[pl.ds(off, HALF)], ia.at[h0], sem),
            pltpu.async_copy(pc.at[pl.ds(off, HALF)], ib.at[h0], sem),
        ]
        second = [
            pltpu.async_copy(pd.at[pl.ds(off + HALF, HALF)], dat.at[h1], sem),
            pltpu.async_copy(pr.at[pl.ds(off + HALF, HALF)], ia.at[h1], sem),
            pltpu.async_copy(pc.at[pl.ds(off + HALF, HALF)], ib.at[h1], sem),
        ]

        for cp in first:
            cp.wait()

        def _pmain(i):
            sl = pl.ds(i, L)
            d = dat[sl]
            va = ia[sl]
            vb = ib[sl]
            g0 = plsc.load_gather(srcA, [vb])
            plsc.addupdate_scatter(acc0, [va], d * g0)
            g1 = plsc.load_gather(srcB, [vb])
            plsc.addupdate_scatter(acc1, [va], d * g1)

        for cp in second:
            cp.wait()
        plsc.parallel_loop(0, CHUNK, L, unroll=4)(_pmain)

        pltpu.sync_copy(acc0, opx.at[s])
        pltpu.sync_copy(acc1, opux.at[s])

    @pl.when(c != 0)
    def _a_side():
        h0 = pl.ds(0, HALF)
        h1 = pl.ds(HALF, HALF)
        first = [
            pltpu.async_copy(y, t1, sem),
            pltpu.async_copy(sv, t2, sem),
            pltpu.async_copy(w.at[pl.ds(N, M)], srcA, sem),
            pltpu.async_copy(w.at[pl.ds(0, N)], srcB, sem),
            pltpu.async_copy(ad.at[pl.ds(off, HALF)], dat.at[h0], sem),
            pltpu.async_copy(ar.at[pl.ds(off, HALF)], ia.at[h0], sem),
            pltpu.async_copy(ac.at[pl.ds(off, HALF)], ib.at[h0], sem),
        ]
        second = [
            pltpu.async_copy(ad.at[pl.ds(off + HALF, HALF)], dat.at[h1], sem),
            pltpu.async_copy(ar.at[pl.ds(off + HALF, HALF)], ia.at[h1], sem),
            pltpu.async_copy(ac.at[pl.ds(off + HALF, HALF)], ib.at[h1], sem),
        ]

        for cp in first:
            cp.wait()

        @plsc.parallel_loop(0, M, L, unroll=4)
        def _uy(i):
            sl = pl.ds(i, L)
            v = t1[sl] - t2[sl]
            srcA[sl] = jnp.where(v > 0, srcA[sl], jnp.zeros((L,), jnp.float32))

        def _amain(i):
            sl = pl.ds(i, L)
            d = dat[sl]
            va = ia[sl]
            vb = ib[sl]
            g0 = plsc.load_gather(srcA, [va])
            plsc.addupdate_scatter(acc0, [vb], d * g0)
            g1 = plsc.load_gather(srcB, [vb])
            plsc.addupdate_scatter(acc1, [va], d * g1)

        for cp in second:
            cp.wait()
        plsc.parallel_loop(0, CHUNK, L, unroll=4)(_amain)

        pltpu.sync_copy(acc0, oatuy.at[s])
        pltpu.sync_copy(acc1, oaux.at[s])


_sc_spmv = functools.partial(
    pl.kernel,
    out_type=[jax.ShapeDtypeStruct((NS, N), jnp.float32) for _ in range(4)],
    mesh=plsc.VectorSubcoreMesh(
        core_axis_name="c", subcore_axis_name="s", num_cores=NC, num_subcores=NS
    ),
    scratch_types=[
        pltpu.VMEM((CHUNK,), jnp.float32),
        pltpu.VMEM((CHUNK,), jnp.int32),
        pltpu.VMEM((CHUNK,), jnp.int32),
        pltpu.VMEM((N,), jnp.float32),
        pltpu.VMEM((N,), jnp.float32),
        pltpu.VMEM((M,), jnp.float32),
        pltpu.VMEM((M,), jnp.float32),
        pltpu.VMEM((N,), jnp.float32),
        pltpu.VMEM((N,), jnp.float32),
        pltpu.SemaphoreType.DMA,
    ],
    compiler_params=pltpu.CompilerParams(needs_layout_passes=False),
)(_sc_body)


def _tc_pre(y_r, s_r, q_r, b_r, w_r, c1, c2, scal):
    wx = w_r[pl.ds(0, N)]
    wy = w_r[pl.ds(N, M)]
    wtau = jnp.sum(w_r[pl.ds(N + M, 1)])
    v = y_r[...] - s_r[...]
    uy = jnp.where(v > 0, wy, jnp.zeros_like(wy))
    c1[...] = q_r[...] * wtau
    c2[...] = b_r[...] * wtau + wy - uy
    sval = -jnp.sum(q_r[...] * wx) - jnp.sum(b_r[...] * uy)
    scal[...] = jnp.full((1,), sval, jnp.float32)


def _tc_post(x_r, w_r, c1, c2, scal,
             pxp, puxp, atuyp, auxp, out):
    Px = jnp.sum(pxp[...], axis=0)
    Pux = jnp.sum(puxp[...], axis=0)
    ATuy = jnp.sum(atuyp[...], axis=0)
    Aux = jnp.sum(auxp[...], axis=0)
    wx = w_r[pl.ds(0, N)]
    wtau = jnp.sum(w_r[pl.ds(N + M, 1)])
    out[pl.ds(0, N)] = Pux + ATuy + c1[...]
    out[pl.ds(N, M)] = -Aux + c2[...]
    xTPx = jnp.sum(x_r[...] * Px)
    r3val = (jnp.sum(scal[...]) - 2.0 * jnp.sum(Px * wx) + xTPx * wtau)
    out[pl.ds(N + M, 1)] = jnp.full((1,), r3val, jnp.float32)


def kernel(x, y, s, q, b, P_data, P_rows, P_cols, A_data, A_rows, A_cols, w):
    px_p, pux_p, atuy_p, aux_p = _sc_spmv(
        P_data, P_rows.astype(jnp.int32), P_cols.astype(jnp.int32),
        A_data, A_rows.astype(jnp.int32), A_cols.astype(jnp.int32),
        x, y, s, w,
    )

    c1, c2, scal = pl.pallas_call(
        _tc_pre,
        out_shape=[
            jax.ShapeDtypeStruct((N,), jnp.float32),
            jax.ShapeDtypeStruct((M,), jnp.float32),
            jax.ShapeDtypeStruct((1,), jnp.float32),
        ],
    )(y, s, q, b, w)

    return pl.pallas_call(
        _tc_post,
        out_shape=jax.ShapeDtypeStruct((N + M + 1,), jnp.float32),
    )(x, w, c1, c2, scal, px_p, pux_p, atuy_p, aux_p)


# speedup vs baseline: 1.0206x; 1.0206x over previous
import functools

import jax
import jax.numpy as jnp
from jax import lax
from jax.experimental import pallas as pl
from jax.experimental.pallas import tpu as pltpu
from jax.experimental.pallas import tpu_sc as plsc

N = 4096
M = 4096
NNZ = 163840

NC = 2
NS = 16
L = 16
CHUNK = NNZ // NS


def _sc_body(pd, pr, pc, ad, ar, ac, x, y, sv, w,
             opx, opux, oatuy, oaux,
             dat, ia, ib, srcA, srcB, t1, t2, acc0, acc1, sem):
    c = lax.axis_index("c")
    s = lax.axis_index("s")
    off = s * CHUNK

    HALF = CHUNK // 2

    @plsc.parallel_loop(0, N, L, unroll=4)
    def _z(i):
        z = jnp.zeros((L,), jnp.float32)
        acc0[pl.ds(i, L)] = z
        acc1[pl.ds(i, L)] = z

    @pl.when(c == 0)
    def _p_side():
        h0 = pl.ds(0, HALF)
        h1 = pl.ds(HALF, HALF)
        first = [
            pltpu.async_copy(x, srcA, sem),
            pltpu.async_copy(w.at[pl.ds(0, N)], srcB, sem),
            pltpu.async_copy(pd.at[pl.ds(off, HALF)], dat.at[h0], sem),
            pltpu.async_copy(pr.at[pl.ds(off, HALF)], ia.at[h0], sem),
            pltpu.async_copy(pc.at[pl.ds(off, HALF)], ib.at[h0], sem),
        ]
        second = [
            pltpu.async_copy(pd.at[pl.ds(off + HALF, HALF)], dat.at[h1], sem),
            pltpu.async_copy(pr.at[pl.ds(off + HALF, HALF)], ia.at[h1], sem),
            pltpu.async_copy(pc.at[pl.ds(off + HALF, HALF)], ib.at[h1], sem),
        ]

        for cp in first:
            cp.wait()

        def _pmain(i):
            sl = pl.ds(i, L)
            d = dat[sl]
            va = ia[sl]
            vb = ib[sl]
            g0 = plsc.load_gather(srcA, [vb])
            plsc.addupdate_scatter(acc0, [va], d * g0)
            g1 = plsc.load_gather(srcB, [vb])
            plsc.addupdate_scatter(acc1, [va], d * g1)

        for cp in second:
            cp.wait()
        plsc.parallel_loop(0, CHUNK, L, unroll=4)(_pmain)

        pltpu.sync_copy(acc0, opx.at[s])
        pltpu.sync_copy(acc1, opux.at[s])

    @pl.when(c != 0)
    def _a_side():
        h0 = pl.ds(0, HALF)
        h1 = pl.ds(HALF, HALF)
        first = [
            pltpu.async_copy(y, t1, sem),
            pltpu.async_copy(sv, t2, sem),
            pltpu.async_copy(w.at[pl.ds(N, M)], srcA, sem),
            pltpu.async_copy(w.at[pl.ds(0, N)], srcB, sem),
            pltpu.async_copy(ad.at[pl.ds(off, HALF)], dat.at[h0], sem),
            pltpu.async_copy(ar.at[pl.ds(off, HALF)], ia.at[h0], sem),
            pltpu.async_copy(ac.at[pl.ds(off, HALF)], ib.at[h0], sem),
        ]
        second = [
            pltpu.async_copy(ad.at[pl.ds(off + HALF, HALF)], dat.at[h1], sem),
            pltpu.async_copy(ar.at[pl.ds(off + HALF, HALF)], ia.at[h1], sem),
            pltpu.async_copy(ac.at[pl.ds(off + HALF, HALF)], ib.at[h1], sem),
        ]

        for cp in first:
            cp.wait()

        @plsc.parallel_loop(0, M, L, unroll=4)
        def _uy(i):
            sl = pl.ds(i, L)
            v = t1[sl] - t2[sl]
            srcA[sl] = jnp.where(v > 0, srcA[sl], jnp.zeros((L,), jnp.float32))

        def _amain(i):
            sl = pl.ds(i, L)
            d = dat[sl]
            va = ia[sl]
            vb = ib[sl]
            g0 = plsc.load_gather(srcA, [va])
            plsc.addupdate_scatter(acc0, [vb], d * g0)
            g1 = plsc.load_gather(srcB, [vb])
            plsc.addupdate_scatter(acc1, [va], d * g1)

        for cp in second:
            cp.wait()
        plsc.parallel_loop(0, CHUNK, L, unroll=4)(_amain)

        pltpu.sync_copy(acc0, oatuy.at[s])
        pltpu.sync_copy(acc1, oaux.at[s])


_sc_spmv = functools.partial(
    pl.kernel,
    out_type=[jax.ShapeDtypeStruct((NS, N), jnp.float32) for _ in range(4)],
    mesh=plsc.VectorSubcoreMesh(
        core_axis_name="c", subcore_axis_name="s", num_cores=NC, num_subcores=NS
    ),
    scratch_types=[
        pltpu.VMEM((CHUNK,), jnp.float32),
        pltpu.VMEM((CHUNK,), jnp.int32),
        pltpu.VMEM((CHUNK,), jnp.int32),
        pltpu.VMEM((N,), jnp.float32),
        pltpu.VMEM((N,), jnp.float32),
        pltpu.VMEM((M,), jnp.float32),
        pltpu.VMEM((M,), jnp.float32),
        pltpu.VMEM((N,), jnp.float32),
        pltpu.VMEM((N,), jnp.float32),
        pltpu.SemaphoreType.DMA,
    ],
    compiler_params=pltpu.CompilerParams(needs_layout_passes=False),
)(_sc_body)


def _tc_body(x_r, y_r, s_r, q_r, b_r, w_r,
             pxp, puxp, atuyp, auxp, out):
    Px = jnp.sum(pxp[...], axis=0)
    Pux = jnp.sum(puxp[...], axis=0)
    ATuy = jnp.sum(atuyp[...], axis=0)
    Aux = jnp.sum(auxp[...], axis=0)
    wx = w_r[pl.ds(0, N)]
    wy = w_r[pl.ds(N, M)]
    wtau = jnp.sum(w_r[pl.ds(N + M, 1)])
    v = y_r[...] - s_r[...]
    uy = jnp.where(v > 0, wy, jnp.zeros_like(wy))
    out[pl.ds(0, N)] = Pux + ATuy + q_r[...] * wtau
    out[pl.ds(N, M)] = -Aux + b_r[...] * wtau + wy - uy
    xTPx = jnp.sum(x_r[...] * Px)
    r3val = (-jnp.sum(q_r[...] * wx) - jnp.sum(b_r[...] * uy)
             - 2.0 * jnp.sum(Px * wx) + xTPx * wtau)
    out[pl.ds(N + M, 1)] = jnp.full((1,), r3val, jnp.float32)


def kernel(x, y, s, q, b, P_data, P_rows, P_cols, A_data, A_rows, A_cols, w):
    px_p, pux_p, atuy_p, aux_p = _sc_spmv(
        P_data, P_rows.astype(jnp.int32), P_cols.astype(jnp.int32),
        A_data, A_rows.astype(jnp.int32), A_cols.astype(jnp.int32),
        x, y, s, w,
    )

    return pl.pallas_call(
        _tc_body,
        out_shape=jax.ShapeDtypeStruct((N + M + 1,), jnp.float32),
    )(x, y, s, q, b, w, px_p, pux_p, atuy_p, aux_p)
